# baseline (speedup 1.0000x reference)
import functools

import jax
import jax.numpy as jnp
from jax import lax
from jax.experimental import pallas as pl
from jax.experimental.pallas import tpu as pltpu

_INT_MIN = -(2 ** 31)


def _sortable_key(x_f32):
    i = lax.bitcast_convert_type(x_f32, jnp.int32)
    mneg = lax.shift_right_arithmetic(i, 31)
    return i ^ (mneg & jnp.int32(0x7FFFFFFF))


def _mask_kernel(score_row_ref, score_col_ref, adj_ref, out_ref,
                 selc_ref, scal_ref, *, k, rb):
    r = pl.program_id(1)

    @pl.when(r == 0)
    def _compute_selection():
        s = score_row_ref[0]
        skey = _sortable_key(s)
        ub = skey ^ jnp.int32(_INT_MIN)

        def body(_, carry):
            p, m, kk, bitv = carry
            prefmatch = (ub & m) == p
            hasbit = (ub & bitv) != jnp.int32(0)
            c1 = jnp.sum((prefmatch & hasbit).astype(jnp.int32))
            take = c1 >= kk
            p = jnp.where(take, p | bitv, p)
            kk = jnp.where(take, kk, kk - c1)
            m = m | bitv
            bitv = lax.shift_right_logical(bitv, 1)
            return p, m, kk, bitv

        p, _, kk, _ = lax.fori_loop(
            0, 32, body,
            (jnp.int32(0), jnp.int32(0), jnp.int32(k), jnp.int32(_INT_MIN)))
        t_skey = p ^ jnp.int32(_INT_MIN)

        eq = skey == t_skey
        idx = lax.broadcasted_iota(jnp.int32, s.shape, 1)

        def body2(_, carry):
            x, bitv = carry
            x_try = x | (bitv - jnp.int32(1))
            cnt = jnp.sum((eq & (idx <= x_try)).astype(jnp.int32))
            x = jnp.where(cnt >= kk, x, x | bitv)
            return x, lax.shift_right_logical(bitv, 1)

        xstar, _ = lax.fori_loop(0, 11, body2, (jnp.int32(0), jnp.int32(1024)))

        selc = ((skey > t_skey) | (eq & (idx <= xstar))).astype(jnp.float32)
        selc_ref[...] = selc
        scal_ref[0] = t_skey
        scal_ref[1] = xstar

    t_skey = scal_ref[0]
    xstar = scal_ref[1]
    skey_r = _sortable_key(score_col_ref[0])
    ridx = lax.broadcasted_iota(jnp.int32, skey_r.shape, 0) + r * rb
    selr = ((skey_r > t_skey)
            | ((skey_r == t_skey) & (ridx <= xstar))).astype(jnp.float32)
    out_ref[0] = adj_ref[0] * jnp.maximum(selr, selc_ref[...])


@jax.jit
def kernel(adj, score):
    b, n, _ = adj.shape
    k = n // 2
    rb = 512
    score_row = score.reshape(b, 1, n)

    grid = (b, n // rb)
    return pl.pallas_call(
        functools.partial(_mask_kernel, k=k, rb=rb),
        grid=grid,
        in_specs=[
            pl.BlockSpec((1, 1, n), lambda bi, ri: (bi, 0, 0)),
            pl.BlockSpec((1, rb, 1), lambda bi, ri: (bi, ri, 0)),
            pl.BlockSpec((1, rb, n), lambda bi, ri: (bi, ri, 0)),
        ],
        out_specs=pl.BlockSpec((1, rb, n), lambda bi, ri: (bi, ri, 0)),
        out_shape=jax.ShapeDtypeStruct((b, n, n), adj.dtype),
        scratch_shapes=[
            pltpu.VMEM((1, n), jnp.float32),
            pltpu.SMEM((2,), jnp.int32),
        ],
    )(score_row, score, adj)


# speedup vs baseline: 1.0168x; 1.0168x over previous
import functools

import jax
import jax.numpy as jnp
from jax import lax
from jax.experimental import pallas as pl
from jax.experimental.pallas import tpu as pltpu

_INT_MIN = -(2 ** 31)


def _sortable_key(x_f32):
    i = lax.bitcast_convert_type(x_f32, jnp.int32)
    mneg = lax.shift_right_arithmetic(i, 31)
    return i ^ (mneg & jnp.int32(0x7FFFFFFF))


def _select_scalars(s, k):
    skey = _sortable_key(s)
    ub = skey ^ jnp.int32(_INT_MIN)

    def body(_, carry):
        p, m, kk, bitv = carry
        prefmatch = (ub & m) == p
        hasbit = (ub & bitv) != jnp.int32(0)
        c1 = jnp.sum((prefmatch & hasbit).astype(jnp.int32))
        take = c1 >= kk
        p = jnp.where(take, p | bitv, p)
        kk = jnp.where(take, kk, kk - c1)
        m = m | bitv
        bitv = lax.shift_right_logical(bitv, 1)
        return p, m, kk, bitv

    p, _, kk, _ = lax.fori_loop(
        0, 32, body,
        (jnp.int32(0), jnp.int32(0), jnp.int32(k), jnp.int32(_INT_MIN)))
    t_skey = p ^ jnp.int32(_INT_MIN)

    eq = skey == t_skey
    idx = lax.broadcasted_iota(jnp.int32, s.shape, 1)

    def body2(_, carry):
        x, bitv = carry
        x_try = x | (bitv - jnp.int32(1))
        cnt = jnp.sum((eq & (idx <= x_try)).astype(jnp.int32))
        x = jnp.where(cnt >= kk, x, x | bitv)
        return x, lax.shift_right_logical(bitv, 1)

    xstar, _ = lax.fori_loop(0, 11, body2, (jnp.int32(0), jnp.int32(1024)))
    return t_skey, xstar, skey, eq, idx


def _mask_kernel(score_row_ref, score_col_ref, adj_ref, out_ref,
                 selc_ref, scal_ref, *, k, rb, nb):
    b = pl.program_id(0)
    r = pl.program_id(1)

    def compute_select(bi, slot):
        s = score_row_ref[bi]
        t_skey, xstar, skey, eq, idx = _select_scalars(s, k)
        selc_ref[slot] = ((skey > t_skey)
                          | (eq & (idx <= xstar))).astype(jnp.float32)
        scal_ref[slot, 0] = t_skey
        scal_ref[slot, 1] = xstar

    @pl.when((b == 0) & (r == 0))
    def _head():
        compute_select(0, 0)

    @pl.when((r == 1) & (b < nb - 1))
    def _ahead():
        compute_select(b + 1, (b + 1) % 2)

    slot = b % 2
    t_skey = scal_ref[slot, 0]
    xstar = scal_ref[slot, 1]
    skey_r = _sortable_key(score_col_ref[0])
    ridx = lax.broadcasted_iota(jnp.int32, skey_r.shape, 0) + r * rb
    selr = ((skey_r > t_skey)
            | ((skey_r == t_skey) & (ridx <= xstar))).astype(jnp.float32)
    out_ref[0] = adj_ref[0] * jnp.maximum(selr, selc_ref[slot])


@jax.jit
def kernel(adj, score):
    b, n, _ = adj.shape
    k = n // 2
    rb = 512
    score_row = score.reshape(b, 1, n)

    grid = (b, n // rb)
    return pl.pallas_call(
        functools.partial(_mask_kernel, k=k, rb=rb, nb=b),
        grid=grid,
        in_specs=[
            pl.BlockSpec((b, 1, n), lambda bi, ri: (0, 0, 0)),
            pl.BlockSpec((1, rb, 1), lambda bi, ri: (bi, ri, 0)),
            pl.BlockSpec((1, rb, n), lambda bi, ri: (bi, ri, 0)),
        ],
        out_specs=pl.BlockSpec((1, rb, n), lambda bi, ri: (bi, ri, 0)),
        out_shape=jax.ShapeDtypeStruct((b, n, n), adj.dtype),
        scratch_shapes=[
            pltpu.VMEM((2, 1, n), jnp.float32),
            pltpu.SMEM((2, 2), jnp.int32),
        ],
    )(score_row, score, adj)


# speedup vs baseline: 1.1492x; 1.1302x over previous
import functools

import jax
import jax.numpy as jnp
from jax import lax
from jax.experimental import pallas as pl
from jax.experimental.pallas import tpu as pltpu

_INT_MIN = -(2 ** 31)


def _sortable_key(x_f32):
    i = lax.bitcast_convert_type(x_f32, jnp.int32)
    mneg = lax.shift_right_arithmetic(i, 31)
    return i ^ (mneg & jnp.int32(0x7FFFFFFF))


def _select_scalars(s, k):
    skey = _sortable_key(s)
    ub = skey ^ jnp.int32(_INT_MIN)

    def body(_, carry):
        p, m, kk, bitv = carry
        prefmatch = (ub & m) == p
        hasbit = (ub & bitv) != jnp.int32(0)
        c1 = jnp.sum((prefmatch & hasbit).astype(jnp.int32))
        take = c1 >= kk
        p = jnp.where(take, p | bitv, p)
        kk = jnp.where(take, kk, kk - c1)
        m = m | bitv
        bitv = lax.shift_right_logical(bitv, 1)
        return p, m, kk, bitv

    p, _, kk, _ = lax.fori_loop(
        0, 32, body,
        (jnp.int32(0), jnp.int32(0), jnp.int32(k), jnp.int32(_INT_MIN)))
    t_skey = p ^ jnp.int32(_INT_MIN)

    eq = skey == t_skey
    idx = lax.broadcasted_iota(jnp.int32, s.shape, 1)

    def body2(_, carry):
        x, bitv = carry
        x_try = x | (bitv - jnp.int32(1))
        cnt = jnp.sum((eq & (idx <= x_try)).astype(jnp.int32))
        x = jnp.where(cnt >= kk, x, x | bitv)
        return x, lax.shift_right_logical(bitv, 1)

    xstar, _ = lax.fori_loop(0, 11, body2, (jnp.int32(0), jnp.int32(1024)))
    return t_skey, xstar, skey, eq, idx


def _mask_kernel(score_row_ref, score_col_ref, adj_ref, out_ref,
                 selc_ref, scal_ref, *, k, rb, nb):
    b = pl.program_id(0)
    r = pl.program_id(1)

    def compute_select(bi, slot):
        s = score_row_ref[bi]
        t_skey, xstar, skey, eq, idx = _select_scalars(s, k)
        selc_ref[slot] = ((skey > t_skey)
                          | (eq & (idx <= xstar))).astype(jnp.float32)
        scal_ref[slot, 0] = t_skey
        scal_ref[slot, 1] = xstar

    @pl.when((b == 0) & (r == 0))
    def _head():
        compute_select(0, 0)

    @pl.when((r == 1) & (b < nb - 1))
    def _ahead():
        compute_select(b + 1, (b + 1) % 2)

    slot = b % 2
    t_skey = scal_ref[slot, 0]
    xstar = scal_ref[slot, 1]
    skey_r = _sortable_key(score_col_ref[0])
    ridx = lax.broadcasted_iota(jnp.int32, skey_r.shape, 0) + r * rb
    selr = ((skey_r > t_skey)
            | ((skey_r == t_skey) & (ridx <= xstar))).astype(jnp.float32)
    out_ref[0] = adj_ref[0] * jnp.maximum(selr, selc_ref[slot])


@jax.jit
def kernel(adj, score):
    b, n, _ = adj.shape
    k = n // 2
    rb = 1024
    score_row = score.reshape(b, 1, n)

    grid = (b, n // rb)
    return pl.pallas_call(
        functools.partial(_mask_kernel, k=k, rb=rb, nb=b),
        grid=grid,
        in_specs=[
            pl.BlockSpec((b, 1, n), lambda bi, ri: (0, 0, 0)),
            pl.BlockSpec((1, rb, 1), lambda bi, ri: (bi, ri, 0)),
            pl.BlockSpec((1, rb, n), lambda bi, ri: (bi, ri, 0)),
        ],
        out_specs=pl.BlockSpec((1, rb, n), lambda bi, ri: (bi, ri, 0)),
        out_shape=jax.ShapeDtypeStruct((b, n, n), adj.dtype),
        scratch_shapes=[
            pltpu.VMEM((2, 1, n), jnp.float32),
            pltpu.SMEM((2, 2), jnp.int32),
        ],
    )(score_row, score, adj)
